# baseline (speedup 1.0000x reference)
import functools

import jax
import jax.numpy as jnp
from jax import lax
from jax.experimental import pallas as pl
from jax.experimental.pallas import tpu as pltpu
from jax.experimental.pallas import tpu_sc as plsc

T = 2048
D = 1024
FF = 2048
E = 8
Z = 2
NE = E + Z
EPAD = 128
TB = 256
NTILES = (T * 2) // TB + E
NROWS = NTILES * TB
R_PAD = NROWS + TB
DUMP = NROWS
NW = 32
TPW = T // NW
CB = 128
NB = T // CB

_NEG = -1e30


def _router_meta_body(x_ref, wr_ref, bias_ref, meta_i_ref, meta_f_ref,
                      tiles_ref, xbf_ref):
    x = x_ref[...]
    xbf_ref[...] = x.astype(jnp.bfloat16)
    logits = lax.dot_general(
        x, wr_ref[...], (((1,), (0,)), ((), ())),
        preferred_element_type=jnp.float32,
        precision=lax.Precision.DEFAULT)
    lane = lax.broadcasted_iota(jnp.int32, (T, EPAD), 1)
    valid = lane < NE
    lm = jnp.where(valid, logits, _NEG)
    mx = jnp.max(lm, axis=1, keepdims=True)
    ex = jnp.where(valid, jnp.exp(lm - mx), 0.0)
    scores = ex / jnp.sum(ex, axis=1, keepdims=True)
    sel = jnp.where(valid, scores + bias_ref[...], _NEG)
    m1 = jnp.max(sel, axis=1, keepdims=True)
    i1 = jnp.min(jnp.where(sel == m1, lane, EPAD), axis=1, keepdims=True)
    sel2 = jnp.where(lane == i1, _NEG, sel)
    m2 = jnp.max(sel2, axis=1, keepdims=True)
    i2 = jnp.min(jnp.where(sel2 == m2, lane, EPAD), axis=1, keepdims=True)
    w0 = jnp.sum(jnp.where(lane == i1, scores, 0.0), axis=1, keepdims=True)
    w1 = jnp.sum(jnp.where(lane == i2, scores, 0.0), axis=1, keepdims=True)
    picked = (lane == i1) | (lane == i2)
    zw = jnp.sum(jnp.where(picked & (lane >= E) & valid, scores, 0.0),
                 axis=1, keepdims=True)

    oh0 = jnp.where(lane == i1, 1.0, 0.0).astype(jnp.bfloat16)
    oh1 = jnp.where(lane == i2, 1.0, 0.0).astype(jnp.bfloat16)
    ri = lax.broadcasted_iota(jnp.int32, (CB, CB), 0)
    ci = lax.broadcasted_iota(jnp.int32, (CB, CB), 1)
    ltri = jnp.where(ri > ci, 1.0, 0.0).astype(jnp.bfloat16)
    dn = (((1,), (0,)), ((), ()))

    def _ranks(oh, run):
        parts = []
        for b in range(NB):
            blk = lax.slice(oh, (b * CB, 0), ((b + 1) * CB, EPAD))
            intra = lax.dot_general(ltri, blk, dn,
                                    preferred_element_type=jnp.float32)
            parts.append(intra + run)
            run = run + jnp.sum(blk.astype(jnp.float32), axis=0,
                                keepdims=True)
        return jnp.concatenate(parts, axis=0), run

    zero_row = jnp.zeros((1, EPAD), jnp.float32)
    rank0, run = _ranks(oh0, zero_row)
    rank1, counts = _ranks(oh1, run)

    lrow = lax.broadcasted_iota(jnp.int32, (1, EPAD), 1)
    tiles = jnp.where(lrow < E,
                      jnp.floor((counts + (TB - 1)) * (1.0 / TB)), 0.0)
    run_t = jnp.zeros((1, 1), jnp.float32)
    offs = jnp.zeros((1, EPAD), jnp.float32)
    te_cnt = jnp.zeros((1, EPAD), jnp.float32)
    ti_f = lrow.astype(jnp.float32)
    for e in range(E):
        offs = offs + jnp.where(lrow == e, run_t * TB, 0.0)
        te_cnt = te_cnt + jnp.where(run_t <= ti_f, 1.0, 0.0)
        run_t = run_t + lax.slice(tiles, (0, e), (1, e + 1))
    total = run_t

    r0 = jnp.sum(jnp.where(lane == i1, rank0, 0.0), axis=1, keepdims=True)
    r1 = jnp.sum(jnp.where(lane == i2, rank1, 0.0), axis=1, keepdims=True)
    off0 = jnp.sum(jnp.where(lane == i1, offs, 0.0), axis=1, keepdims=True)
    off1 = jnp.sum(jnp.where(lane == i2, offs, 0.0), axis=1, keepdims=True)
    routed0 = i1 < E
    routed1 = i2 < E
    pos0 = jnp.where(routed0, off0 + r0, jnp.float32(DUMP))
    pos1 = jnp.where(routed1, off1 + r1, jnp.float32(DUMP))

    meta_i = (jnp.where(lane == 0, pos0, 0.0)
              + jnp.where(lane == 1, pos1, 0.0)
              + jnp.where(lane == 2, jnp.where(routed0, 1.0, 0.0), 0.0)
              + jnp.where(lane == 3, jnp.where(routed1, 1.0, 0.0), 0.0))
    meta_i_ref[...] = meta_i.astype(jnp.int32)
    grp = lane // 16
    meta_f_ref[...] = (jnp.where(grp == 0, w0, 0.0)
                       + jnp.where(grp == 1, w1, 0.0)
                       + jnp.where(grp == 2, zw, 0.0)
                       + jnp.where(grp == 3,
                                   jnp.where(routed0, 1.0, 0.0), 0.0)
                       + jnp.where(grp == 4,
                                   jnp.where(routed1, 1.0, 0.0), 0.0))

    te = jnp.minimum(jnp.maximum(te_cnt - 1.0, 0.0), E - 1)
    xi = jnp.where(lrow == NTILES, float(NTILES),
                   jnp.minimum(ti_f, jnp.maximum(total - 1.0, 0.0)))
    tv = jnp.where(ti_f < total, 1.0, 0.0)
    srow = lax.broadcasted_iota(jnp.int32, (8, EPAD), 0)
    tiles_meta = (jnp.where(srow == 0, te, 0.0)
                  + jnp.where(srow == 1, xi, 0.0)
                  + jnp.where(srow == 2, tv, 0.0))
    tiles_ref[...] = tiles_meta.astype(jnp.int32)


@functools.cache
def _make_scatter_x():
    mesh = plsc.VectorSubcoreMesh(core_axis_name="c", subcore_axis_name="s")

    @functools.partial(
        pl.kernel, mesh=mesh,
        out_type=jax.ShapeDtypeStruct((R_PAD, D // 2), jnp.int32),
        scratch_types=[
            pltpu.VMEM((32,), jnp.int32),
            pltpu.VMEM((32,), jnp.int32),
            pltpu.VMEM((32, D // 2), jnp.int32),
            pltpu.VMEM((32, D // 2), jnp.int32),
            pltpu.SemaphoreType.DMA,
            pltpu.SemaphoreType.DMA,
        ])
    def _scatter_x(x_hbm, pos_hbm, xs_hbm, idx_v0, idx_v1, rows_v0,
                   rows_v1, seml, sems):
        wid = lax.axis_index("s") * 2 + lax.axis_index("c")
        tbase = (wid % 16) * 128
        idx = (idx_v0, idx_v1)
        rows = (rows_v0, rows_v1)
        ld = pltpu.async_copy(x_hbm.at[pl.ds(tbase, 32)], rows_v0, seml)
        pltpu.sync_copy(pos_hbm.at[wid, 0], idx_v0)
        st = [None, None]
        for c in range(4):
            if c + 1 < 4:
                if st[(c + 1) % 2] is not None:
                    st[(c + 1) % 2].wait()
                nld = pltpu.async_copy(
                    x_hbm.at[pl.ds(tbase + (c + 1) * 32, 32)],
                    rows[(c + 1) % 2], seml)
                pltpu.sync_copy(pos_hbm.at[wid, c + 1], idx[(c + 1) % 2])
            ld.wait()
            st[c % 2] = pltpu.async_copy(rows[c % 2], xs_hbm.at[idx[c % 2]],
                                         sems)
            if c + 1 < 4:
                ld = nld
        st[0].wait()
        st[1].wait()

    return _scatter_x


def _ffn_body(te_ref, xi_ref, tv_ref, xs_ref, w1_ref, w3_ref, w2_ref, y_ref):
    i = pl.program_id(0)

    @pl.when(i == NTILES)
    def _zero_dump():
        y_ref[...] = jnp.zeros((TB, D), jnp.float32)

    @pl.when(tv_ref[i] != 0)
    def _():
        xb = xs_ref[...]
        dn = (((1,), (0,)), ((), ()))
        acc = jnp.zeros((TB, D), jnp.float32)
        for f in range(2):
            w1c = w1_ref[0, :, f * (FF // 2):(f + 1) * (FF // 2)]
            w3c = w3_ref[0, :, f * (FF // 2):(f + 1) * (FF // 2)]
            w2c = w2_ref[0, f * (FF // 2):(f + 1) * (FF // 2), :]
            a = lax.dot_general(xb, w1c.astype(jnp.bfloat16), dn,
                                preferred_element_type=jnp.float32)
            b = lax.dot_general(xb, w3c.astype(jnp.bfloat16), dn,
                                preferred_element_type=jnp.float32)
            h = (a * (1.0 / (1.0 + jnp.exp(-a))) * b).astype(jnp.bfloat16)
            acc = acc + lax.dot_general(h, w2c.astype(jnp.bfloat16), dn,
                                        preferred_element_type=jnp.float32)
        y_ref[...] = acc


@functools.cache
def _make_combine():
    mesh = plsc.VectorSubcoreMesh(core_axis_name="c", subcore_axis_name="s")

    @functools.partial(
        pl.kernel, mesh=mesh,
        out_type=jax.ShapeDtypeStruct((T, D), jnp.float32),
        scratch_types=[
            pltpu.VMEM((TPW,), jnp.int32),
            pltpu.VMEM((TPW,), jnp.int32),
            pltpu.VMEM((16, EPAD), jnp.float32),
            pltpu.VMEM((16, EPAD), jnp.float32),
            pltpu.VMEM((16, D), jnp.float32),
            pltpu.VMEM((16, D), jnp.float32),
            pltpu.VMEM((16, D), jnp.float32),
            pltpu.VMEM((16, D), jnp.float32),
            pltpu.VMEM((16, D), jnp.float32),
            pltpu.VMEM((16, D), jnp.float32),
            pltpu.VMEM((16, D), jnp.float32),
            pltpu.SemaphoreType.DMA,
            pltpu.SemaphoreType.DMA,
        ])
    def _combine(x_hbm, y_hbm, p0_hbm, p1_hbm, mf_hbm, out_hbm, p0_v,
                 p1_v, mf_v0, mf_v1, y0a, y0b, y1a, y1b, xa, xb, o_v,
                 semg, semw):
        wid = lax.axis_index("s") * 2 + lax.axis_index("c")
        base = wid * TPW
        pltpu.sync_copy(p0_hbm.at[pl.ds(base, TPW)], p0_v)
        pltpu.sync_copy(p1_hbm.at[pl.ds(base, TPW)], p1_v)
        mf = (mf_v0, mf_v1)
        y0 = (y0a, y0b)
        y1 = (y1a, y1b)
        xv = (xa, xb)

        def start(c):
            p = c % 2
            idx0 = p0_v[pl.ds(c * 16, 16)]
            idx1 = p1_v[pl.ds(c * 16, 16)]
            return (pltpu.async_copy(y_hbm.at[idx0], y0[p], semg),
                    pltpu.async_copy(y_hbm.at[idx1], y1[p], semg),
                    pltpu.async_copy(x_hbm.at[pl.ds(base + c * 16, 16)],
                                     xv[p], semg),
                    pltpu.async_copy(mf_hbm.at[pl.ds(base + c * 16, 16)],
                                     mf[p], semg))

        cps = start(0)
        stw = None
        for c in range(4):
            p = c % 2
            if c + 1 < 4:
                ncps = start(c + 1)
            for cp in cps:
                cp.wait()
            if stw is not None:
                stw.wait()
            for i in range(16):
                w0s = mf[p][i, pl.ds(0, 16)]
                w1s = mf[p][i, pl.ds(16, 16)]
                zws = mf[p][i, pl.ds(32, 16)]

                def body(g, _):
                    sl = pl.ds(g * 16, 16)
                    o_v[i, sl] = (w0s * y0[p][i, sl] + w1s * y1[p][i, sl]
                                  + zws * xv[p][i, sl])
                    return 0

                lax.fori_loop(0, D // 16, body, 0)
            stw = pltpu.async_copy(o_v, out_hbm.at[pl.ds(base + c * 16, 16)],
                                   semw)
            if c + 1 < 4:
                cps = ncps
        stw.wait()

    return _combine


@jax.jit
def kernel(hidden_states, Wr, e_score_correction_bias, W1, W3, W2):
    wr_pad = jnp.zeros((D, EPAD), jnp.float32).at[:, :NE].set(Wr)
    bias_pad = jnp.zeros((1, EPAD), jnp.float32).at[0, :NE].set(
        e_score_correction_bias)

    meta_i, meta_f, tiles_meta, xbf = pl.pallas_call(
        _router_meta_body,
        out_shape=(
            jax.ShapeDtypeStruct((T, EPAD), jnp.int32),
            jax.ShapeDtypeStruct((T, EPAD), jnp.float32),
            jax.ShapeDtypeStruct((8, EPAD), jnp.int32),
            jax.ShapeDtypeStruct((T, D), jnp.bfloat16),
        ),
    )(hidden_states, wr_pad, bias_pad)

    pos0 = meta_i[:, 0]
    pos1 = meta_i[:, 1]
    te = tiles_meta[0, :]
    xi = tiles_meta[1, :]
    tv = tiles_meta[2, :]
    pos_all = jnp.concatenate([pos0, pos1]).reshape(NW, 4, 32)

    xbf_i = lax.bitcast_convert_type(xbf.reshape(T, D // 2, 2), jnp.int32)
    xs_i = _make_scatter_x()(xbf_i, pos_all)
    xs = lax.bitcast_convert_type(xs_i, jnp.bfloat16).reshape(R_PAD, D)

    y = pl.pallas_call(
        _ffn_body,
        grid_spec=pltpu.PrefetchScalarGridSpec(
            num_scalar_prefetch=3,
            grid=(NTILES + 1,),
            in_specs=[
                pl.BlockSpec((TB, D), lambda i, te, xi, tv: (xi[i], 0)),
                pl.BlockSpec((1, D, FF), lambda i, te, xi, tv: (te[i], 0, 0)),
                pl.BlockSpec((1, D, FF), lambda i, te, xi, tv: (te[i], 0, 0)),
                pl.BlockSpec((1, FF, D), lambda i, te, xi, tv: (te[i], 0, 0)),
            ],
            out_specs=pl.BlockSpec((TB, D), lambda i, te, xi, tv: (xi[i], 0)),
        ),
        out_shape=jax.ShapeDtypeStruct((R_PAD, D), jnp.float32),
    )(te, xi, tv, xs, W1, W3, W2)

    out = _make_combine()(hidden_states, y, pos0, pos1, meta_f)
    return out


# speedup vs baseline: 1.6593x; 1.6593x over previous
import functools

import jax
import jax.numpy as jnp
from jax import lax
from jax.experimental import pallas as pl
from jax.experimental.pallas import tpu as pltpu
from jax.experimental.pallas import tpu_sc as plsc

T = 2048
D = 1024
FF = 2048
E = 8
Z = 2
NE = E + Z
EPAD = 128
TB = 256
NTILES = (T * 2) // TB + E
NROWS = NTILES * TB
R_PAD = NROWS + TB
DUMP = NROWS
NW = 32
TPW = T // NW
CB = 128
NB = T // CB

_NEG = -1e30


def _router_meta_body(x_ref, wr_ref, bias_ref, meta_i_ref, meta_f_ref,
                      tiles_ref):
    x = x_ref[...]
    logits = lax.dot_general(
        x, wr_ref[...], (((1,), (0,)), ((), ())),
        preferred_element_type=jnp.float32,
        precision=lax.Precision.DEFAULT)
    lane = lax.broadcasted_iota(jnp.int32, (T, EPAD), 1)
    valid = lane < NE
    lm = jnp.where(valid, logits, _NEG)
    mx = jnp.max(lm, axis=1, keepdims=True)
    ex = jnp.where(valid, jnp.exp(lm - mx), 0.0)
    scores = ex / jnp.sum(ex, axis=1, keepdims=True)
    sel = jnp.where(valid, scores + bias_ref[...], _NEG)
    m1 = jnp.max(sel, axis=1, keepdims=True)
    i1 = jnp.min(jnp.where(sel == m1, lane, EPAD), axis=1, keepdims=True)
    sel2 = jnp.where(lane == i1, _NEG, sel)
    m2 = jnp.max(sel2, axis=1, keepdims=True)
    i2 = jnp.min(jnp.where(sel2 == m2, lane, EPAD), axis=1, keepdims=True)
    w0 = jnp.sum(jnp.where(lane == i1, scores, 0.0), axis=1, keepdims=True)
    w1 = jnp.sum(jnp.where(lane == i2, scores, 0.0), axis=1, keepdims=True)
    picked = (lane == i1) | (lane == i2)
    zw = jnp.sum(jnp.where(picked & (lane >= E) & valid, scores, 0.0),
                 axis=1, keepdims=True)

    oh0 = jnp.where(lane == i1, 1.0, 0.0).astype(jnp.bfloat16)
    oh1 = jnp.where(lane == i2, 1.0, 0.0).astype(jnp.bfloat16)
    ri = lax.broadcasted_iota(jnp.int32, (CB, CB), 0)
    ci = lax.broadcasted_iota(jnp.int32, (CB, CB), 1)
    ltri = jnp.where(ri > ci, 1.0, 0.0).astype(jnp.bfloat16)
    dn = (((1,), (0,)), ((), ()))

    def _ranks(oh, run):
        parts = []
        for b in range(NB):
            blk = lax.slice(oh, (b * CB, 0), ((b + 1) * CB, EPAD))
            intra = lax.dot_general(ltri, blk, dn,
                                    preferred_element_type=jnp.float32)
            parts.append(intra + run)
            run = run + jnp.sum(blk.astype(jnp.float32), axis=0,
                                keepdims=True)
        return jnp.concatenate(parts, axis=0), run

    zero_row = jnp.zeros((1, EPAD), jnp.float32)
    rank0, run = _ranks(oh0, zero_row)
    rank1, counts = _ranks(oh1, run)

    lrow = lax.broadcasted_iota(jnp.int32, (1, EPAD), 1)
    tiles = jnp.where(lrow < E,
                      jnp.floor((counts + (TB - 1)) * (1.0 / TB)), 0.0)
    run_t = jnp.zeros((1, 1), jnp.float32)
    offs = jnp.zeros((1, EPAD), jnp.float32)
    te_cnt = jnp.zeros((1, EPAD), jnp.float32)
    ti_f = lrow.astype(jnp.float32)
    for e in range(E):
        offs = offs + jnp.where(lrow == e, run_t * TB, 0.0)
        te_cnt = te_cnt + jnp.where(run_t <= ti_f, 1.0, 0.0)
        run_t = run_t + lax.slice(tiles, (0, e), (1, e + 1))
    total = run_t

    r0 = jnp.sum(jnp.where(lane == i1, rank0, 0.0), axis=1, keepdims=True)
    r1 = jnp.sum(jnp.where(lane == i2, rank1, 0.0), axis=1, keepdims=True)
    off0 = jnp.sum(jnp.where(lane == i1, offs, 0.0), axis=1, keepdims=True)
    off1 = jnp.sum(jnp.where(lane == i2, offs, 0.0), axis=1, keepdims=True)
    routed0 = i1 < E
    routed1 = i2 < E
    pos0 = jnp.where(routed0, off0 + r0, jnp.float32(DUMP))
    pos1 = jnp.where(routed1, off1 + r1, jnp.float32(DUMP))

    meta_i = (jnp.where(lane == 0, pos0, 0.0)
              + jnp.where(lane == 1, pos1, 0.0)
              + jnp.where(lane == 2, jnp.where(routed0, 1.0, 0.0), 0.0)
              + jnp.where(lane == 3, jnp.where(routed1, 1.0, 0.0), 0.0))
    meta_i_ref[...] = meta_i.astype(jnp.int32)
    grp = lane // 16
    meta_f_ref[...] = (jnp.where(grp == 0, w0, 0.0)
                       + jnp.where(grp == 1, w1, 0.0)
                       + jnp.where(grp == 2, zw, 0.0)
                       + jnp.where(grp == 3,
                                   jnp.where(routed0, 1.0, 0.0), 0.0)
                       + jnp.where(grp == 4,
                                   jnp.where(routed1, 1.0, 0.0), 0.0))

    te = jnp.minimum(jnp.maximum(te_cnt - 1.0, 0.0), E - 1)
    xi = jnp.where(lrow == NTILES, float(NTILES),
                   jnp.minimum(ti_f, jnp.maximum(total - 1.0, 0.0)))
    tv = jnp.where(ti_f < total, 1.0, 0.0)
    srow = lax.broadcasted_iota(jnp.int32, (8, EPAD), 0)
    tiles_meta = (jnp.where(srow == 0, te, 0.0)
                  + jnp.where(srow == 1, xi, 0.0)
                  + jnp.where(srow == 2, tv, 0.0))
    tiles_ref[...] = tiles_meta.astype(jnp.int32)


@functools.cache
def _make_scatter_x():
    mesh = plsc.VectorSubcoreMesh(core_axis_name="c", subcore_axis_name="s")

    @functools.partial(
        pl.kernel, mesh=mesh,
        out_type=jax.ShapeDtypeStruct((R_PAD, D), jnp.float32),
        scratch_types=[
            pltpu.VMEM((32,), jnp.int32),
            pltpu.VMEM((32,), jnp.int32),
            pltpu.VMEM((32, D), jnp.float32),
            pltpu.VMEM((32, D), jnp.float32),
            pltpu.SemaphoreType.DMA,
            pltpu.SemaphoreType.DMA,
        ])
    def _scatter_x(x_hbm, pos_hbm, xs_hbm, idx_v0, idx_v1, rows_v0,
                   rows_v1, seml, sems):
        wid = lax.axis_index("s") * 2 + lax.axis_index("c")
        tbase = (wid % 16) * 128
        idx = (idx_v0, idx_v1)
        rows = (rows_v0, rows_v1)
        ld = pltpu.async_copy(x_hbm.at[pl.ds(tbase, 32)], rows_v0, seml)
        pltpu.sync_copy(pos_hbm.at[wid, 0], idx_v0)
        st = [None, None]
        for c in range(4):
            if c + 1 < 4:
                if st[(c + 1) % 2] is not None:
                    st[(c + 1) % 2].wait()
                nld = pltpu.async_copy(
                    x_hbm.at[pl.ds(tbase + (c + 1) * 32, 32)],
                    rows[(c + 1) % 2], seml)
                pltpu.sync_copy(pos_hbm.at[wid, c + 1], idx[(c + 1) % 2])
            ld.wait()
            st[c % 2] = pltpu.async_copy(rows[c % 2], xs_hbm.at[idx[c % 2]],
                                         sems)
            if c + 1 < 4:
                ld = nld
        st[0].wait()
        st[1].wait()

    return _scatter_x


def _ffn_body(te_ref, xi_ref, tv_ref, xs_ref, w1_ref, w3_ref, w2_ref, y_ref):
    i = pl.program_id(0)

    @pl.when(i == NTILES)
    def _zero_dump():
        y_ref[...] = jnp.zeros((TB, D), jnp.float32)

    @pl.when(tv_ref[i] != 0)
    def _():
        xb = xs_ref[...].astype(jnp.bfloat16)
        dn = (((1,), (0,)), ((), ()))
        acc = jnp.zeros((TB, D), jnp.float32)
        for f in range(2):
            w1c = w1_ref[0, :, f * (FF // 2):(f + 1) * (FF // 2)]
            w3c = w3_ref[0, :, f * (FF // 2):(f + 1) * (FF // 2)]
            w2c = w2_ref[0, f * (FF // 2):(f + 1) * (FF // 2), :]
            a = lax.dot_general(xb, w1c.astype(jnp.bfloat16), dn,
                                preferred_element_type=jnp.float32)
            b = lax.dot_general(xb, w3c.astype(jnp.bfloat16), dn,
                                preferred_element_type=jnp.float32)
            h = (a * (1.0 / (1.0 + jnp.exp(-a))) * b).astype(jnp.bfloat16)
            acc = acc + lax.dot_general(h, w2c.astype(jnp.bfloat16), dn,
                                        preferred_element_type=jnp.float32)
        y_ref[...] = acc


@functools.cache
def _make_combine():
    mesh = plsc.VectorSubcoreMesh(core_axis_name="c", subcore_axis_name="s")

    @functools.partial(
        pl.kernel, mesh=mesh,
        out_type=jax.ShapeDtypeStruct((T, D), jnp.float32),
        scratch_types=[
            pltpu.VMEM((TPW,), jnp.int32),
            pltpu.VMEM((TPW,), jnp.int32),
            pltpu.VMEM((16, EPAD), jnp.float32),
            pltpu.VMEM((16, EPAD), jnp.float32),
            pltpu.VMEM((16, D), jnp.float32),
            pltpu.VMEM((16, D), jnp.float32),
            pltpu.VMEM((16, D), jnp.float32),
            pltpu.VMEM((16, D), jnp.float32),
            pltpu.VMEM((16, D), jnp.float32),
            pltpu.VMEM((16, D), jnp.float32),
            pltpu.VMEM((16, D), jnp.float32),
            pltpu.SemaphoreType.DMA,
            pltpu.SemaphoreType.DMA,
        ])
    def _combine(x_hbm, y_hbm, p0_hbm, p1_hbm, mf_hbm, out_hbm, p0_v,
                 p1_v, mf_v0, mf_v1, y0a, y0b, y1a, y1b, xa, xb, o_v,
                 semg, semw):
        wid = lax.axis_index("s") * 2 + lax.axis_index("c")
        base = wid * TPW
        pltpu.sync_copy(p0_hbm.at[pl.ds(base, TPW)], p0_v)
        pltpu.sync_copy(p1_hbm.at[pl.ds(base, TPW)], p1_v)
        mf = (mf_v0, mf_v1)
        y0 = (y0a, y0b)
        y1 = (y1a, y1b)
        xv = (xa, xb)

        def start(c):
            p = c % 2
            idx0 = p0_v[pl.ds(c * 16, 16)]
            idx1 = p1_v[pl.ds(c * 16, 16)]
            return (pltpu.async_copy(y_hbm.at[idx0], y0[p], semg),
                    pltpu.async_copy(y_hbm.at[idx1], y1[p], semg),
                    pltpu.async_copy(x_hbm.at[pl.ds(base + c * 16, 16)],
                                     xv[p], semg),
                    pltpu.async_copy(mf_hbm.at[pl.ds(base + c * 16, 16)],
                                     mf[p], semg))

        cps = start(0)
        stw = None
        for c in range(4):
            p = c % 2
            if c + 1 < 4:
                ncps = start(c + 1)
            for cp in cps:
                cp.wait()
            if stw is not None:
                stw.wait()
            for i in range(16):
                w0s = mf[p][i, pl.ds(0, 16)]
                w1s = mf[p][i, pl.ds(16, 16)]
                zws = mf[p][i, pl.ds(32, 16)]

                def body(g, _):
                    sl = pl.ds(g * 16, 16)
                    o_v[i, sl] = (w0s * y0[p][i, sl] + w1s * y1[p][i, sl]
                                  + zws * xv[p][i, sl])
                    return 0

                lax.fori_loop(0, D // 16, body, 0, unroll=4)
            stw = pltpu.async_copy(o_v, out_hbm.at[pl.ds(base + c * 16, 16)],
                                   semw)
            if c + 1 < 4:
                cps = ncps
        stw.wait()

    return _combine


@jax.jit
def kernel(hidden_states, Wr, e_score_correction_bias, W1, W3, W2):
    wr_pad = jnp.zeros((D, EPAD), jnp.float32).at[:, :NE].set(Wr)
    bias_pad = jnp.zeros((1, EPAD), jnp.float32).at[0, :NE].set(
        e_score_correction_bias)

    meta_i, meta_f, tiles_meta = pl.pallas_call(
        _router_meta_body,
        out_shape=(
            jax.ShapeDtypeStruct((T, EPAD), jnp.int32),
            jax.ShapeDtypeStruct((T, EPAD), jnp.float32),
            jax.ShapeDtypeStruct((8, EPAD), jnp.int32),
        ),
    )(hidden_states, wr_pad, bias_pad)

    pos0 = meta_i[:, 0]
    pos1 = meta_i[:, 1]
    te = tiles_meta[0, :]
    xi = tiles_meta[1, :]
    tv = tiles_meta[2, :]
    pos_all = jnp.concatenate([pos0, pos1]).reshape(NW, 4, 32)

    xs = _make_scatter_x()(hidden_states, pos_all)

    y = pl.pallas_call(
        _ffn_body,
        grid_spec=pltpu.PrefetchScalarGridSpec(
            num_scalar_prefetch=3,
            grid=(NTILES + 1,),
            in_specs=[
                pl.BlockSpec((TB, D), lambda i, te, xi, tv: (xi[i], 0)),
                pl.BlockSpec((1, D, FF), lambda i, te, xi, tv: (te[i], 0, 0)),
                pl.BlockSpec((1, D, FF), lambda i, te, xi, tv: (te[i], 0, 0)),
                pl.BlockSpec((1, FF, D), lambda i, te, xi, tv: (te[i], 0, 0)),
            ],
            out_specs=pl.BlockSpec((TB, D), lambda i, te, xi, tv: (xi[i], 0)),
        ),
        out_shape=jax.ShapeDtypeStruct((R_PAD, D), jnp.float32),
    )(te, xi, tv, xs, W1, W3, W2)

    out = _make_combine()(hidden_states, y, pos0, pos1, meta_f)
    return out


# speedup vs baseline: 1.6893x; 1.0181x over previous
import functools

import jax
import jax.numpy as jnp
from jax import lax
from jax.experimental import pallas as pl
from jax.experimental.pallas import tpu as pltpu
from jax.experimental.pallas import tpu_sc as plsc

T = 2048
D = 1024
FF = 2048
E = 8
Z = 2
NE = E + Z
EPAD = 128
TB = 256
NTILES = (T * 2) // TB + E
NROWS = NTILES * TB
R_PAD = NROWS + TB
DUMP = NROWS
NW = 32
TPW = T // NW
CB = 128
NB = T // CB

_NEG = -1e30


def _router_meta_body(x_ref, wr_ref, bias_ref, meta_i_ref, meta_f_ref,
                      tiles_ref):
    x = x_ref[...]
    logits = lax.dot_general(
        x, wr_ref[...], (((1,), (0,)), ((), ())),
        preferred_element_type=jnp.float32,
        precision=lax.Precision.DEFAULT)
    lane = lax.broadcasted_iota(jnp.int32, (T, EPAD), 1)
    valid = lane < NE
    lm = jnp.where(valid, logits, _NEG)
    mx = jnp.max(lm, axis=1, keepdims=True)
    ex = jnp.where(valid, jnp.exp(lm - mx), 0.0)
    scores = ex / jnp.sum(ex, axis=1, keepdims=True)
    sel = jnp.where(valid, scores + bias_ref[...], _NEG)
    m1 = jnp.max(sel, axis=1, keepdims=True)
    i1 = jnp.min(jnp.where(sel == m1, lane, EPAD), axis=1, keepdims=True)
    sel2 = jnp.where(lane == i1, _NEG, sel)
    m2 = jnp.max(sel2, axis=1, keepdims=True)
    i2 = jnp.min(jnp.where(sel2 == m2, lane, EPAD), axis=1, keepdims=True)
    w0 = jnp.sum(jnp.where(lane == i1, scores, 0.0), axis=1, keepdims=True)
    w1 = jnp.sum(jnp.where(lane == i2, scores, 0.0), axis=1, keepdims=True)
    picked = (lane == i1) | (lane == i2)
    zw = jnp.sum(jnp.where(picked & (lane >= E) & valid, scores, 0.0),
                 axis=1, keepdims=True)

    oh0 = jnp.where(lane == i1, 1.0, 0.0).astype(jnp.bfloat16)
    oh1 = jnp.where(lane == i2, 1.0, 0.0).astype(jnp.bfloat16)
    ri = lax.broadcasted_iota(jnp.int32, (CB, CB), 0)
    ci = lax.broadcasted_iota(jnp.int32, (CB, CB), 1)
    ltri = jnp.where(ri > ci, 1.0, 0.0).astype(jnp.bfloat16)
    dn = (((1,), (0,)), ((), ()))

    def _ranks(oh, run):
        parts = []
        for b in range(NB):
            blk = lax.slice(oh, (b * CB, 0), ((b + 1) * CB, EPAD))
            intra = lax.dot_general(ltri, blk, dn,
                                    preferred_element_type=jnp.float32)
            parts.append(intra + run)
            run = run + jnp.sum(blk.astype(jnp.float32), axis=0,
                                keepdims=True)
        return jnp.concatenate(parts, axis=0), run

    zero_row = jnp.zeros((1, EPAD), jnp.float32)
    rank0, run = _ranks(oh0, zero_row)
    rank1, counts = _ranks(oh1, run)

    lrow = lax.broadcasted_iota(jnp.int32, (1, EPAD), 1)
    tiles = jnp.where(lrow < E,
                      jnp.floor((counts + (TB - 1)) * (1.0 / TB)), 0.0)
    run_t = jnp.zeros((1, 1), jnp.float32)
    offs = jnp.zeros((1, EPAD), jnp.float32)
    te_cnt = jnp.zeros((1, EPAD), jnp.float32)
    ti_f = lrow.astype(jnp.float32)
    for e in range(E):
        offs = offs + jnp.where(lrow == e, run_t * TB, 0.0)
        te_cnt = te_cnt + jnp.where(run_t <= ti_f, 1.0, 0.0)
        run_t = run_t + lax.slice(tiles, (0, e), (1, e + 1))
    total = run_t

    r0 = jnp.sum(jnp.where(lane == i1, rank0, 0.0), axis=1, keepdims=True)
    r1 = jnp.sum(jnp.where(lane == i2, rank1, 0.0), axis=1, keepdims=True)
    off0 = jnp.sum(jnp.where(lane == i1, offs, 0.0), axis=1, keepdims=True)
    off1 = jnp.sum(jnp.where(lane == i2, offs, 0.0), axis=1, keepdims=True)
    routed0 = i1 < E
    routed1 = i2 < E
    pos0 = jnp.where(routed0, off0 + r0, jnp.float32(DUMP))
    pos1 = jnp.where(routed1, off1 + r1, jnp.float32(DUMP))

    meta_i = (jnp.where(lane == 0, pos0, 0.0)
              + jnp.where(lane == 1, pos1, 0.0)
              + jnp.where(lane == 2, jnp.where(routed0, 1.0, 0.0), 0.0)
              + jnp.where(lane == 3, jnp.where(routed1, 1.0, 0.0), 0.0))
    meta_i_ref[...] = meta_i.astype(jnp.int32)
    grp = lane // 16
    meta_f_ref[...] = (jnp.where(grp == 0, w0, 0.0)
                       + jnp.where(grp == 1, w1, 0.0)
                       + jnp.where(grp == 2, zw, 0.0)
                       + jnp.where(grp == 3,
                                   jnp.where(routed0, 1.0, 0.0), 0.0)
                       + jnp.where(grp == 4,
                                   jnp.where(routed1, 1.0, 0.0), 0.0))

    te = jnp.minimum(jnp.maximum(te_cnt - 1.0, 0.0), E - 1)
    xi = jnp.where(lrow == NTILES, float(NTILES),
                   jnp.minimum(ti_f, jnp.maximum(total - 1.0, 0.0)))
    tv = jnp.where(ti_f < total, 1.0, 0.0)
    srow = lax.broadcasted_iota(jnp.int32, (8, EPAD), 0)
    tiles_meta = (jnp.where(srow == 0, te, 0.0)
                  + jnp.where(srow == 1, xi, 0.0)
                  + jnp.where(srow == 2, tv, 0.0))
    tiles_ref[...] = tiles_meta.astype(jnp.int32)


@functools.cache
def _make_scatter_x():
    mesh = plsc.VectorSubcoreMesh(core_axis_name="c", subcore_axis_name="s")

    @functools.partial(
        pl.kernel, mesh=mesh,
        out_type=jax.ShapeDtypeStruct((R_PAD, D), jnp.float32),
        scratch_types=[
            pltpu.VMEM((32,), jnp.int32),
            pltpu.VMEM((32,), jnp.int32),
            pltpu.VMEM((32, D), jnp.float32),
            pltpu.VMEM((32, D), jnp.float32),
            pltpu.SemaphoreType.DMA,
            pltpu.SemaphoreType.DMA,
        ])
    def _scatter_x(x_hbm, pos_hbm, xs_hbm, idx_v0, idx_v1, rows_v0,
                   rows_v1, seml, sems):
        wid = lax.axis_index("s") * 2 + lax.axis_index("c")
        tbase = (wid % 16) * 128
        idx = (idx_v0, idx_v1)
        rows = (rows_v0, rows_v1)
        ld = pltpu.async_copy(x_hbm.at[pl.ds(tbase, 32)], rows_v0, seml)
        pltpu.sync_copy(pos_hbm.at[wid, 0], idx_v0)
        st = [None, None]
        for c in range(4):
            if c + 1 < 4:
                if st[(c + 1) % 2] is not None:
                    st[(c + 1) % 2].wait()
                nld = pltpu.async_copy(
                    x_hbm.at[pl.ds(tbase + (c + 1) * 32, 32)],
                    rows[(c + 1) % 2], seml)
                pltpu.sync_copy(pos_hbm.at[wid, c + 1], idx[(c + 1) % 2])
            ld.wait()
            st[c % 2] = pltpu.async_copy(rows[c % 2], xs_hbm.at[idx[c % 2]],
                                         sems)
            if c + 1 < 4:
                ld = nld
        st[0].wait()
        st[1].wait()

    return _scatter_x


def _ffn_body(te_ref, xi_ref, tv_ref, xs_ref, w1_ref, w3_ref, w2_ref, y_ref):
    i = pl.program_id(0)

    @pl.when(i == NTILES)
    def _zero_dump():
        y_ref[...] = jnp.zeros((TB, D), jnp.float32)

    @pl.when(tv_ref[i] != 0)
    def _():
        xb = xs_ref[...].astype(jnp.bfloat16)
        dn = (((1,), (0,)), ((), ()))
        acc = jnp.zeros((TB, D), jnp.float32)
        for f in range(2):
            w1c = w1_ref[0, :, f * (FF // 2):(f + 1) * (FF // 2)]
            w3c = w3_ref[0, :, f * (FF // 2):(f + 1) * (FF // 2)]
            w2c = w2_ref[0, f * (FF // 2):(f + 1) * (FF // 2), :]
            a = lax.dot_general(xb, w1c.astype(jnp.bfloat16), dn,
                                preferred_element_type=jnp.float32)
            b = lax.dot_general(xb, w3c.astype(jnp.bfloat16), dn,
                                preferred_element_type=jnp.float32)
            h = (a * (1.0 / (1.0 + jnp.exp(-a))) * b).astype(jnp.bfloat16)
            acc = acc + lax.dot_general(h, w2c.astype(jnp.bfloat16), dn,
                                        preferred_element_type=jnp.float32)
        y_ref[...] = acc


@functools.cache
def _make_combine():
    mesh = plsc.VectorSubcoreMesh(core_axis_name="c", subcore_axis_name="s")

    @functools.partial(
        pl.kernel, mesh=mesh,
        out_type=jax.ShapeDtypeStruct((T, D), jnp.float32),
        scratch_types=[
            pltpu.VMEM((TPW,), jnp.int32),
            pltpu.VMEM((TPW,), jnp.int32),
            pltpu.VMEM((16, EPAD), jnp.float32),
            pltpu.VMEM((16, EPAD), jnp.float32),
            pltpu.VMEM((16, D), jnp.float32),
            pltpu.VMEM((16, D), jnp.float32),
            pltpu.VMEM((16, D), jnp.float32),
            pltpu.VMEM((16, D), jnp.float32),
            pltpu.VMEM((16, D), jnp.float32),
            pltpu.VMEM((16, D), jnp.float32),
            pltpu.VMEM((16, D), jnp.float32),
            pltpu.SemaphoreType.DMA,
            pltpu.SemaphoreType.DMA,
        ])
    def _combine(x_hbm, y_hbm, p0_hbm, p1_hbm, mf_hbm, out_hbm, p0_v,
                 p1_v, mf_v0, mf_v1, y0a, y0b, y1a, y1b, xa, xb, o_v,
                 semg, semw):
        wid = lax.axis_index("s") * 2 + lax.axis_index("c")
        base = wid * TPW
        pltpu.sync_copy(p0_hbm.at[pl.ds(base, TPW)], p0_v)
        pltpu.sync_copy(p1_hbm.at[pl.ds(base, TPW)], p1_v)
        mf = (mf_v0, mf_v1)
        y0 = (y0a, y0b)
        y1 = (y1a, y1b)
        xv = (xa, xb)

        def start(c):
            p = c % 2
            idx0 = p0_v[pl.ds(c * 16, 16)]
            idx1 = p1_v[pl.ds(c * 16, 16)]
            return (pltpu.async_copy(y_hbm.at[idx0], y0[p], semg),
                    pltpu.async_copy(y_hbm.at[idx1], y1[p], semg),
                    pltpu.async_copy(x_hbm.at[pl.ds(base + c * 16, 16)],
                                     xv[p], semg),
                    pltpu.async_copy(mf_hbm.at[pl.ds(base + c * 16, 16)],
                                     mf[p], semg))

        cps = start(0)
        stw = None
        for c in range(4):
            p = c % 2
            if c + 1 < 4:
                ncps = start(c + 1)
            for cp in cps:
                cp.wait()
            if stw is not None:
                stw.wait()
            for i in range(16):
                w0s = mf[p][i, pl.ds(0, 16)]
                w1s = mf[p][i, pl.ds(16, 16)]
                zws = mf[p][i, pl.ds(32, 16)]

                def body(g, _):
                    sl = pl.ds(g * 16, 16)
                    o_v[i, sl] = (w0s * y0[p][i, sl] + w1s * y1[p][i, sl]
                                  + zws * xv[p][i, sl])
                    return 0

                lax.fori_loop(0, D // 16, body, 0)
            stw = pltpu.async_copy(o_v, out_hbm.at[pl.ds(base + c * 16, 16)],
                                   semw)
            if c + 1 < 4:
                cps = ncps
        stw.wait()

    return _combine


@jax.jit
def kernel(hidden_states, Wr, e_score_correction_bias, W1, W3, W2):
    wr_pad = jnp.zeros((D, EPAD), jnp.float32).at[:, :NE].set(Wr)
    bias_pad = jnp.zeros((1, EPAD), jnp.float32).at[0, :NE].set(
        e_score_correction_bias)

    meta_i, meta_f, tiles_meta = pl.pallas_call(
        _router_meta_body,
        out_shape=(
            jax.ShapeDtypeStruct((T, EPAD), jnp.int32),
            jax.ShapeDtypeStruct((T, EPAD), jnp.float32),
            jax.ShapeDtypeStruct((8, EPAD), jnp.int32),
        ),
    )(hidden_states, wr_pad, bias_pad)

    pos0 = meta_i[:, 0]
    pos1 = meta_i[:, 1]
    te = tiles_meta[0, :]
    xi = tiles_meta[1, :]
    tv = tiles_meta[2, :]
    pos_all = jnp.concatenate([pos0, pos1]).reshape(NW, 4, 32)

    xs = _make_scatter_x()(hidden_states, pos_all)

    y = pl.pallas_call(
        _ffn_body,
        grid_spec=pltpu.PrefetchScalarGridSpec(
            num_scalar_prefetch=3,
            grid=(NTILES + 1,),
            in_specs=[
                pl.BlockSpec((TB, D), lambda i, te, xi, tv: (xi[i], 0)),
                pl.BlockSpec((1, D, FF), lambda i, te, xi, tv: (te[i], 0, 0)),
                pl.BlockSpec((1, D, FF), lambda i, te, xi, tv: (te[i], 0, 0)),
                pl.BlockSpec((1, FF, D), lambda i, te, xi, tv: (te[i], 0, 0)),
            ],
            out_specs=pl.BlockSpec((TB, D), lambda i, te, xi, tv: (xi[i], 0)),
        ),
        out_shape=jax.ShapeDtypeStruct((R_PAD, D), jnp.float32),
    )(te, xi, tv, xs, W1, W3, W2)

    out = _make_combine()(hidden_states, y, pos0, pos1, meta_f)
    return out
